# baseline (speedup 1.0000x reference)
import functools

import jax
import jax.numpy as jnp
from jax import lax
from jax.experimental import pallas as pl
from jax.experimental.pallas import tpu as pltpu
from jax.experimental.pallas import tpu_sc as plsc

N_TOK = 8192
D_MODEL = 2048
N_EXP = 64
LANES = 16
N_WORKERS = 32
TPW = N_TOK // N_WORKERS
TOK_BLK = 1024


def _logits_body(hs_ref, w_ref, out_ref):
    out_ref[...] = lax.dot_general(
        hs_ref[...], w_ref[...],
        dimension_numbers=(((1,), (1,)), ((), ())),
        preferred_element_type=jnp.float32,
    )


def _compute_logits(hidden_states, weight):
    return pl.pallas_call(
        _logits_body,
        grid=(N_TOK // TOK_BLK,),
        in_specs=[
            pl.BlockSpec((TOK_BLK, D_MODEL), lambda i: (i, 0)),
            pl.BlockSpec((N_EXP, D_MODEL), lambda i: (0, 0)),
        ],
        out_specs=pl.BlockSpec((TOK_BLK, N_EXP), lambda i: (i, 0)),
        out_shape=jax.ShapeDtypeStruct((N_TOK, N_EXP), jnp.float32),
    )(hidden_states, weight)


_SC_MESH = plsc.VectorSubcoreMesh(core_axis_name="c", subcore_axis_name="s")


@functools.partial(
    pl.kernel,
    out_type=(
        jax.ShapeDtypeStruct((N_TOK, 2), jnp.int32),
        jax.ShapeDtypeStruct((N_TOK, 2), jnp.float32),
    ),
    mesh=_SC_MESH,
    compiler_params=pltpu.CompilerParams(needs_layout_passes=False),
    scratch_types=[
        pltpu.VMEM((TPW, N_EXP), jnp.float32),
        pltpu.VMEM((TPW, 2), jnp.int32),
        pltpu.VMEM((TPW, 2), jnp.float32),
    ],
)
def _route_sc(logits_hbm, idx_hbm, w_hbm, ltile, idxv, wv):
    wid = lax.axis_index("s") * 2 + lax.axis_index("c")
    base = wid * TPW
    pltpu.sync_copy(logits_hbm.at[pl.ds(base, TPW), :], ltile)

    lanes = lax.iota(jnp.int32, LANES)
    col0 = jnp.zeros((LANES,), jnp.int32)
    col1 = col0 + 1

    def per_group(g, carry):
        tok = g * LANES + lanes
        neg = jnp.full((LANES,), -jnp.inf, jnp.float32)
        zero = jnp.zeros((LANES,), jnp.int32)

        def per_expert(e, c):
            m1, i1, m2, i2 = c
            ev = jnp.zeros((LANES,), jnp.int32) + e
            v = plsc.load_gather(ltile, [tok, ev])
            gt1 = v > m1
            gt2 = v > m2
            nm2 = jnp.where(gt1, m1, jnp.where(gt2, v, m2))
            ni2 = jnp.where(gt1, i1, jnp.where(gt2, ev, i2))
            nm1 = jnp.where(gt1, v, m1)
            ni1 = jnp.where(gt1, ev, i1)
            return nm1, ni1, nm2, ni2

        m1, i1, m2, i2 = lax.fori_loop(
            0, N_EXP, per_expert, (neg, zero, neg, zero))

        ed = jnp.exp(m2 - m1)
        s = ed + 1.0
        w1 = 1.0 / s
        w2 = ed / s
        plsc.store_scatter(idxv, [tok, col0], i1)
        plsc.store_scatter(idxv, [tok, col1], i2)
        plsc.store_scatter(wv, [tok, col0], w1)
        plsc.store_scatter(wv, [tok, col1], w2)
        return carry

    lax.fori_loop(0, TPW // LANES, per_group, 0)

    pltpu.sync_copy(idxv, idx_hbm.at[pl.ds(base, TPW), :])
    pltpu.sync_copy(wv, w_hbm.at[pl.ds(base, TPW), :])


def kernel(hidden_states, weight):
    logits = _compute_logits(hidden_states, weight)
    topk_idx, topk_weight = _route_sc(logits)
    return topk_idx, topk_weight


# speedup vs baseline: 1.2140x; 1.2140x over previous
import functools

import jax
import jax.numpy as jnp
from jax import lax
from jax.experimental import pallas as pl
from jax.experimental.pallas import tpu as pltpu
from jax.experimental.pallas import tpu_sc as plsc

N_TOK = 8192
D_MODEL = 2048
N_EXP = 64
LANES = 16
N_WORKERS = 32
TPW = N_TOK // N_WORKERS
TOK_BLK = 1024


def _logits_body(w_ref, hs_ref, out_ref):
    out_ref[...] = lax.dot_general(
        w_ref[...], hs_ref[...],
        dimension_numbers=(((1,), (1,)), ((), ())),
        preferred_element_type=jnp.float32,
    )


def _compute_logits_t(hidden_states, weight):
    return pl.pallas_call(
        _logits_body,
        grid=(N_TOK // TOK_BLK,),
        in_specs=[
            pl.BlockSpec((N_EXP, D_MODEL), lambda i: (0, 0)),
            pl.BlockSpec((TOK_BLK, D_MODEL), lambda i: (i, 0)),
        ],
        out_specs=pl.BlockSpec((N_EXP, TOK_BLK), lambda i: (0, i)),
        out_shape=jax.ShapeDtypeStruct((N_EXP, N_TOK), jnp.float32),
    )(weight, hidden_states)


_SC_MESH = plsc.VectorSubcoreMesh(core_axis_name="c", subcore_axis_name="s")


def _top2_stream(ltile, off):
    m1 = ltile[0, pl.ds(off, LANES)]
    i1 = jnp.zeros((LANES,), jnp.int32)
    m2 = jnp.full((LANES,), -jnp.inf, jnp.float32)
    i2 = i1
    for e in range(1, N_EXP):
        v = ltile[e, pl.ds(off, LANES)]
        ev = jnp.full((LANES,), e, jnp.int32)
        gt1 = v > m1
        gt2 = v > m2
        nm2 = jnp.where(gt1, m1, jnp.where(gt2, v, m2))
        ni2 = jnp.where(gt1, i1, jnp.where(gt2, ev, i2))
        m1 = jnp.where(gt1, v, m1)
        i1 = jnp.where(gt1, ev, i1)
        m2 = nm2
        i2 = ni2
    return m1, i1, m2, i2


@functools.partial(
    pl.kernel,
    out_type=(
        jax.ShapeDtypeStruct((N_TOK, 2), jnp.int32),
        jax.ShapeDtypeStruct((N_TOK, 2), jnp.float32),
    ),
    mesh=_SC_MESH,
    compiler_params=pltpu.CompilerParams(needs_layout_passes=False),
    scratch_types=[
        pltpu.VMEM((N_EXP, TPW), jnp.float32),
        pltpu.VMEM((TPW, 2), jnp.int32),
        pltpu.VMEM((TPW, 2), jnp.float32),
    ],
)
def _route_sc(logits_hbm, idx_hbm, w_hbm, ltile, idxv, wv):
    wid = lax.axis_index("s") * 2 + lax.axis_index("c")
    base = wid * TPW
    pltpu.sync_copy(logits_hbm.at[:, pl.ds(base, TPW)], ltile)

    lanes = lax.iota(jnp.int32, LANES)
    col0 = jnp.zeros((LANES,), jnp.int32)
    col1 = col0 + 1

    def emit(tok, m1, i1, m2, i2):
        ed = jnp.exp(m2 - m1)
        s = ed + 1.0
        plsc.store_scatter(idxv, [tok, col0], i1)
        plsc.store_scatter(idxv, [tok, col1], i2)
        plsc.store_scatter(wv, [tok, col0], 1.0 / s)
        plsc.store_scatter(wv, [tok, col1], ed / s)

    def per_pair(p, carry):
        off_a = p * (2 * LANES)
        off_b = off_a + LANES
        ra = _top2_stream(ltile, off_a)
        rb = _top2_stream(ltile, off_b)
        emit(off_a + lanes, *ra)
        emit(off_b + lanes, *rb)
        return carry

    lax.fori_loop(0, TPW // (2 * LANES), per_pair, 0)

    pltpu.sync_copy(idxv, idx_hbm.at[pl.ds(base, TPW), :])
    pltpu.sync_copy(wv, w_hbm.at[pl.ds(base, TPW), :])


def kernel(hidden_states, weight):
    logits_t = _compute_logits_t(hidden_states, weight)
    topk_idx, topk_weight = _route_sc(logits_t)
    return topk_idx, topk_weight
